# baseline (speedup 1.0000x reference)
import functools

import jax
import jax.numpy as jnp
from jax import lax
from jax.experimental import pallas as pl
from jax.experimental.pallas import tpu as pltpu
from jax.experimental.pallas import tpu_sc as plsc

BATCH = 16384
HIDDEN = 64
B_CONST = 3.2

_info = plsc.get_sparse_core_info()
_NC, _NS = _info.num_cores, _info.num_subcores
NW = _NC * _NS
ROWS_W = BATCH // NW
CHUNK = 128
NCH = ROWS_W // CHUNK


def _body(u_hbm, i_hbm, P_hbm, Q_hbm, bu_hbm, bi_hbm, out_hbm,
          u_v, i_v, pu_v, qi_v, bu_v, bi_v, sred_v, out_v, sem):
    wid = lax.axis_index("s") * _NC + lax.axis_index("c")
    base = wid * ROWS_W

    pltpu.sync_copy(u_hbm.at[wid], u_v)
    pltpu.sync_copy(i_hbm.at[wid], i_v)

    copies = []
    for j in range(NCH):
        sl = pl.ds(j * CHUNK, CHUNK)
        copies.append(pltpu.async_copy(P_hbm.at[u_v.at[j]], pu_v.at[sl], sem))
        copies.append(pltpu.async_copy(Q_hbm.at[i_v.at[j]], qi_v.at[sl], sem))
        copies.append(pltpu.async_copy(bu_hbm.at[u_v.at[j]], bu_v.at[sl], sem))
        copies.append(pltpu.async_copy(bi_hbm.at[i_v.at[j]], bi_v.at[sl], sem))
    for c in copies:
        c.wait()

    lanes = lax.iota(jnp.int32, 16) * ROWS_W

    def prow(rr, carry):
        for k in range(4):
            r = rr * 4 + k
            s = None
            for c in range(4):
                sl = pl.ds(c * 16, 16)
                pq = pu_v[r, sl] * qi_v[r, sl]
                s = pq if s is None else s + pq
            plsc.store_scatter(sred_v, [lanes + r], s)
        return carry

    lax.fori_loop(0, ROWS_W // 4, prow, 0)

    def group(g, carry):
        off = pl.multiple_of(g * 16, 16)
        acc0 = bu_v[pl.ds(off, 16)] + bi_v[pl.ds(off, 16)] + B_CONST
        acc1 = jnp.zeros((16,), jnp.float32)
        acc2 = jnp.zeros((16,), jnp.float32)
        acc3 = jnp.zeros((16,), jnp.float32)
        accs = [acc0, acc1, acc2, acc3]
        for l in range(16):
            accs[l % 4] = accs[l % 4] + sred_v[pl.ds(off + l * ROWS_W, 16)]
        out_v[pl.ds(off, 16)] = (accs[0] + accs[1]) + (accs[2] + accs[3])
        return carry

    lax.fori_loop(0, ROWS_W // 16, group, 0)

    pltpu.sync_copy(out_v, out_hbm.at[pl.ds(base, ROWS_W)])


@jax.jit
def _autorec(u, i, P, Q, b_u, b_i):
    mesh = plsc.VectorSubcoreMesh(core_axis_name="c", subcore_axis_name="s")
    kern = functools.partial(
        pl.kernel,
        mesh=mesh,
        out_type=jax.ShapeDtypeStruct((BATCH,), jnp.float32),
        scratch_types=[
            pltpu.VMEM((NCH, CHUNK), jnp.int32),
            pltpu.VMEM((NCH, CHUNK), jnp.int32),
            pltpu.VMEM((ROWS_W, HIDDEN), jnp.float32),
            pltpu.VMEM((ROWS_W, HIDDEN), jnp.float32),
            pltpu.VMEM((ROWS_W,), jnp.float32),
            pltpu.VMEM((ROWS_W,), jnp.float32),
            pltpu.VMEM((16 * ROWS_W,), jnp.float32),
            pltpu.VMEM((ROWS_W,), jnp.float32),
            pltpu.SemaphoreType.DMA,
        ],
        compiler_params=pltpu.CompilerParams(
            needs_layout_passes=False, use_tc_tiling_on_sc=False),
    )(_body)
    return kern(u, i, P, Q, b_u, b_i)


def kernel(rows, P, Q, b_u, b_i):
    u = rows[:, 0].reshape(NW, NCH, CHUNK)
    i = rows[:, 1].reshape(NW, NCH, CHUNK)
    return _autorec(u, i, P, Q, b_u, b_i)
